# baseline (speedup 1.0000x reference)
import functools

import jax
import jax.numpy as jnp
from jax import lax
from jax.experimental import pallas as pl
from jax.experimental.pallas import tpu as pltpu
from jax.experimental.pallas import tpu_sc as plsc

N = 10000
E = 320000
NSLAB = 32
B = 128
NBLK = 79
EPS = 10000
ACC_N = 10240
RPT = ACC_N // 16
DC = 64
WA = DC + 16

_mesh = plsc.VectorSubcoreMesh(core_axis_name="c", subcore_axis_name="s")
_params = pltpu.CompilerParams(
    needs_layout_passes=False, use_tc_tiling_on_sc=False)


def _make_sc_edge(col_split):
    @functools.partial(
        pl.kernel,
        out_type=jax.ShapeDtypeStruct((2, ACC_N, WA), jnp.float32),
        mesh=_mesh,
        compiler_params=_params,
        scratch_types=[
            pltpu.VMEM((NBLK, B), jnp.int32),
            pltpu.VMEM((NBLK, B), jnp.int32),
            pltpu.VMEM((N,), jnp.float32),
            pltpu.VMEM((N,), jnp.float32),
            pltpu.VMEM((B, DC), jnp.float32),
            pltpu.VMEM((B, DC), jnp.float32),
            pltpu.VMEM((B, WA), jnp.float32),
            pltpu.VMEM((B, WA), jnp.float32),
            pltpu.VMEM((B,), jnp.float32),
            pltpu.VMEM_SHARED((ACC_N, WA), jnp.float32),
            pltpu.SemaphoreType.DMA,
            pltpu.SemaphoreType.DMA,
            pltpu.SemaphoreType.DMA,
            pltpu.SemaphoreType.DMA,
        ],
    )
    def sc_edge(ht_hbm, hs_hbm, hd_hbm, src_hbm, dst_hbm, out_hbm,
                srcv, dstv, hsv, hdv, grows0, grows1, rows0, rows1, exv,
                acc, semg0, semg1, sems0, sems1):
        c = lax.axis_index("c")
        s = lax.axis_index("s")

        pltpu.sync_copy(hs_hbm, hsv)
        pltpu.sync_copy(hd_hbm, hdv)

        z16 = jnp.zeros((16,), jnp.float32)
        cpr = WA // 16

        def zb(i, carry):
            r = i // cpr
            cc = i % cpr
            rows0[r, pl.ds(cc * 16, 16)] = z16
            return carry
        lax.fori_loop(0, B * cpr, zb, 0)

        def zcp(j, carry):
            pltpu.sync_copy(rows0, acc.at[pl.ds(s * RPT + j * B, B)])
            return carry
        lax.fori_loop(0, RPT // B, zcp, 0)

        plsc.subcore_barrier()

        lane = lax.iota(jnp.int32, 16)
        nslab_per_tile = 2 if col_split else 1

        for j in range(nslab_per_tile):
            if col_split:
                slab = s * 2 + j
                hsrc = ht_hbm.at[c]
            else:
                slab = c * 16 + s
                hsrc = ht_hbm
            pltpu.sync_copy(src_hbm.at[slab], srcv)
            pltpu.sync_copy(dst_hbm.at[slab], dstv)

            def issue_g(b, gbuf, sem):
                pltpu.async_copy(hsrc.at[srcv.at[b]], gbuf, sem)

            def wait_g(b, gbuf, sem):
                pltpu.make_async_copy(hsrc.at[srcv.at[b]], gbuf, sem).wait()

            def issue_s(b, rbuf, sem):
                pltpu.async_copy(rbuf, acc.at[dstv.at[b]], sem, add=True)

            def wait_s(b, rbuf, sem):
                pltpu.make_async_copy(rbuf, acc.at[dstv.at[b]], sem).wait()

            def compute(b, gbuf, rbuf):
                def grpf(g, cy):
                    s16 = srcv[b, pl.ds(g * 16, 16)]
                    d16 = dstv[b, pl.ds(g * 16, 16)]
                    hs16 = plsc.load_gather(hsv, [s16])
                    hd16 = plsc.load_gather(hdv, [d16])
                    e = hs16 + hd16
                    e = jnp.where(e >= 0.0, e, 0.2 * e)
                    eid = b * B + g * 16 + lane
                    ex = jnp.where(eid < EPS, jnp.exp(e), 0.0)
                    exv[pl.ds(g * 16, 16)] = ex
                    for u in range(16):
                        k = g * 16 + u
                        exk = plsc.load_gather(
                            exv, [jnp.full((16,), k, jnp.int32)])
                        for cc in range(DC // 16):
                            rbuf[k, pl.ds(cc * 16, 16)] = (
                                gbuf[k, pl.ds(cc * 16, 16)] * exk)
                        rbuf[k, pl.ds(DC, 16)] = jnp.where(
                            lane == 0, exk, 0.0)
                    return cy
                lax.fori_loop(0, B // 16, grpf, 0)

            issue_g(0, grows0, semg0)

            def pairf(p, carry):
                b0 = 2 * p
                b1 = b0 + 1
                wait_g(b0, grows0, semg0)
                issue_g(b1, grows1, semg1)

                @pl.when(p > 0)
                def _():
                    wait_s(b0 - 2, rows0, sems0)
                compute(b0, grows0, rows0)
                issue_s(b0, rows0, sems0)

                wait_g(b1, grows1, semg1)
                issue_g(b0 + 2, grows0, semg0)

                @pl.when(p > 0)
                def _():
                    wait_s(b1 - 2, rows1, sems1)
                compute(b1, grows1, rows1)
                issue_s(b1, rows1, sems1)
                return carry
            lax.fori_loop(0, (NBLK - 1) // 2, pairf, 0)

            bt = NBLK - 1
            wait_g(bt, grows0, semg0)
            wait_s(bt - 2, rows0, sems0)
            compute(bt, grows0, rows0)
            issue_s(bt, rows0, sems0)
            wait_s(bt, rows0, sems0)
            wait_s(bt - 1, rows1, sems1)

        plsc.subcore_barrier()

        def wb(j, carry):
            r0 = s * RPT + j * B
            pltpu.sync_copy(acc.at[pl.ds(r0, B)],
                            out_hbm.at[c, pl.ds(r0, B)])
            return carry
        lax.fori_loop(0, RPT // B, wb, 0)

    return sc_edge


_sc_edge_csplit = _make_sc_edge(True)
_sc_edge_esplit = _make_sc_edge(False)


def _tc_first(x, W, a_s, a_d):
    n, _ = x.shape
    D = W.shape[1]

    def body(x_ref, w_ref, as_ref, ad_ref,
             h_ref, hs_ref, hd_ref, ns_ref, ds_ref):
        h = jnp.dot(x_ref[...], w_ref[...], preferred_element_type=jnp.float32)
        hs = jnp.dot(h, as_ref[...], preferred_element_type=jnp.float32)
        hd = jnp.dot(h, ad_ref[...], preferred_element_type=jnp.float32)
        e = hs + hd
        e = jnp.where(e >= 0.0, e, 0.2 * e)
        exs = jnp.exp(e)
        h_ref[0] = h[:, :DC]
        h_ref[1] = h[:, DC:]
        hs_ref[...] = hs
        hd_ref[...] = hd
        ns_ref[...] = exs * h
        ds_ref[...] = exs

    return pl.pallas_call(
        body,
        out_shape=(
            jax.ShapeDtypeStruct((2, n, DC), jnp.float32),
            jax.ShapeDtypeStruct((n, 1), jnp.float32),
            jax.ShapeDtypeStruct((n, 1), jnp.float32),
            jax.ShapeDtypeStruct((n, D), jnp.float32),
            jax.ShapeDtypeStruct((n, 1), jnp.float32),
        ),
    )(x, W, a_s, a_d)


def _combine(part_ref, nums_ref, dens_ref, n, Dp, col_split):
    if col_split:
        num = (jnp.concatenate(
            [part_ref[0, :n, :DC], part_ref[1, :n, :DC]], axis=1)
            + nums_ref[...])
        den = part_ref[0, :n, DC:DC + 1] + dens_ref[...]
    else:
        num = part_ref[0, :n, :Dp] + part_ref[1, :n, :Dp] + nums_ref[...]
        den = (part_ref[0, :n, Dp:Dp + 1] + part_ref[1, :n, Dp:Dp + 1]
               + dens_ref[...])
    return num, den


def _tc_mid(part, nums, dens, b_prev, W, a_s, a_d, col_split):
    n, Dp = nums.shape
    D = W.shape[1]

    def body(part_ref, nums_ref, dens_ref, b_ref, w_ref, as_ref, ad_ref,
             h_ref, hs_ref, hd_ref, ns_ref, ds_ref):
        num, den = _combine(part_ref, nums_ref, dens_ref, n, Dp, col_split)
        prev = num / (den + 1e-16) + b_ref[...]
        prev = jnp.maximum(prev, 0.0)
        h = jnp.dot(prev, w_ref[...], preferred_element_type=jnp.float32)
        hs = jnp.dot(h, as_ref[...], preferred_element_type=jnp.float32)
        hd = jnp.dot(h, ad_ref[...], preferred_element_type=jnp.float32)
        e = hs + hd
        e = jnp.where(e >= 0.0, e, 0.2 * e)
        exs = jnp.exp(e)
        h_ref[...] = h
        hs_ref[...] = hs
        hd_ref[...] = hd
        ns_ref[...] = exs * h
        ds_ref[...] = exs

    return pl.pallas_call(
        body,
        out_shape=(
            jax.ShapeDtypeStruct((n, D), jnp.float32),
            jax.ShapeDtypeStruct((n, 1), jnp.float32),
            jax.ShapeDtypeStruct((n, 1), jnp.float32),
            jax.ShapeDtypeStruct((n, D), jnp.float32),
            jax.ShapeDtypeStruct((n, 1), jnp.float32),
        ),
    )(part, nums, dens, b_prev, W, a_s, a_d)


def _tc_final(part, nums, dens, b, td, cs):
    n, Dp = nums.shape

    def body(part_ref, nums_ref, dens_ref, b_ref, td_ref, cs_ref, out_ref):
        num, den = _combine(part_ref, nums_ref, dens_ref, n, Dp, False)
        o = num / (den + 1e-16) + b_ref[...]
        o = o * (td_ref[0, 0] * cs_ref[0, 0])
        out_ref[...] = jnp.mean(o, axis=0, keepdims=True)

    return pl.pallas_call(
        body,
        out_shape=jax.ShapeDtypeStruct((1, Dp), jnp.float32),
    )(part, nums, dens, b, td, cs)


def kernel(x, edge_index, W1, a1_src, a1_dst, b1, W2, a2_src, a2_dst, b2,
           W3, a3_src, a3_dst, b3, threshold_distance, cansu):
    pad = NBLK * B - EPS
    src = jnp.pad(edge_index[0].reshape(NSLAB, EPS),
                  ((0, 0), (0, pad))).reshape(NSLAB, NBLK, B)
    dst = jnp.pad(edge_index[1].reshape(NSLAB, EPS),
                  ((0, 0), (0, pad))).reshape(NSLAB, NBLK, B)
    td = threshold_distance.reshape(1, 1)
    cs = cansu.reshape(1, 1)

    h1, hs1, hd1, ns1, ds1 = _tc_first(
        x, W1, a1_src.reshape(-1, 1), a1_dst.reshape(-1, 1))
    part1 = _sc_edge_csplit(h1, hs1.reshape(-1), hd1.reshape(-1), src, dst)

    h2, hs2, hd2, ns2, ds2 = _tc_mid(
        part1, ns1, ds1, b1.reshape(1, -1),
        W2, a2_src.reshape(-1, 1), a2_dst.reshape(-1, 1), col_split=True)
    part2 = _sc_edge_esplit(h2, hs2.reshape(-1), hd2.reshape(-1), src, dst)

    h3, hs3, hd3, ns3, ds3 = _tc_mid(
        part2, ns2, ds2, b2.reshape(1, -1),
        W3, a3_src.reshape(-1, 1), a3_dst.reshape(-1, 1), col_split=False)
    part3 = _sc_edge_esplit(h3, hs3.reshape(-1), hd3.reshape(-1), src, dst)

    return _tc_final(part3, ns3, ds3, b3.reshape(1, -1), td, cs)


# speedup vs baseline: 1.0935x; 1.0935x over previous
import functools

import jax
import jax.numpy as jnp
from jax import lax
from jax.experimental import pallas as pl
from jax.experimental.pallas import tpu as pltpu
from jax.experimental.pallas import tpu_sc as plsc

N = 10000
E = 320000
NSLAB = 32
B = 64
NBLK = 157
EPS = 10000
ACC_N = 10240
RPT = ACC_N // 16
DC = 64
WA = DC + 16

_mesh = plsc.VectorSubcoreMesh(core_axis_name="c", subcore_axis_name="s")
_params = pltpu.CompilerParams(
    needs_layout_passes=False, use_tc_tiling_on_sc=False)


def _make_sc_edge(col_split):
    @functools.partial(
        pl.kernel,
        out_type=jax.ShapeDtypeStruct((2, ACC_N, WA), jnp.float32),
        mesh=_mesh,
        compiler_params=_params,
        scratch_types=[
            pltpu.VMEM((NBLK, B), jnp.int32),
            pltpu.VMEM((NBLK, B), jnp.int32),
            pltpu.VMEM((N,), jnp.float32),
            pltpu.VMEM((N,), jnp.float32),
            pltpu.VMEM((B, DC), jnp.float32),
            pltpu.VMEM((B, DC), jnp.float32),
            pltpu.VMEM((B, WA), jnp.float32),
            pltpu.VMEM((B, WA), jnp.float32),
            pltpu.VMEM((B,), jnp.float32),
            pltpu.VMEM_SHARED((ACC_N, WA), jnp.float32),
            pltpu.SemaphoreType.DMA,
            pltpu.SemaphoreType.DMA,
            pltpu.SemaphoreType.DMA,
            pltpu.SemaphoreType.DMA,
        ],
    )
    def sc_edge(ht_hbm, hs_hbm, hd_hbm, src_hbm, dst_hbm, out_hbm,
                srcv, dstv, hsv, hdv, grows0, grows1, rows0, rows1, exv,
                acc, semg0, semg1, sems0, sems1):
        c = lax.axis_index("c")
        s = lax.axis_index("s")

        pltpu.sync_copy(hs_hbm, hsv)
        pltpu.sync_copy(hd_hbm, hdv)

        z16 = jnp.zeros((16,), jnp.float32)
        cpr = WA // 16

        def zb(i, carry):
            r = i // cpr
            cc = i % cpr
            rows0[r, pl.ds(cc * 16, 16)] = z16
            return carry
        lax.fori_loop(0, B * cpr, zb, 0)

        def zcp(j, carry):
            pltpu.sync_copy(rows0, acc.at[pl.ds(s * RPT + j * B, B)])
            return carry
        lax.fori_loop(0, RPT // B, zcp, 0)

        plsc.subcore_barrier()

        lane = lax.iota(jnp.int32, 16)
        nslab_per_tile = 2 if col_split else 1

        for j in range(nslab_per_tile):
            if col_split:
                slab = s * 2 + j
                hsrc = ht_hbm.at[c]
            else:
                slab = c * 16 + s
                hsrc = ht_hbm
            pltpu.sync_copy(src_hbm.at[slab], srcv)
            pltpu.sync_copy(dst_hbm.at[slab], dstv)

            def issue_g(b, gbuf, sem):
                pltpu.async_copy(hsrc.at[srcv.at[b]], gbuf, sem)

            def wait_g(b, gbuf, sem):
                pltpu.make_async_copy(hsrc.at[srcv.at[b]], gbuf, sem).wait()

            def issue_s(b, rbuf, sem):
                pltpu.async_copy(rbuf, acc.at[dstv.at[b]], sem, add=True)

            def wait_s(b, rbuf, sem):
                pltpu.make_async_copy(rbuf, acc.at[dstv.at[b]], sem).wait()

            def compute(b, gbuf, rbuf):
                def grpf(g, cy):
                    s16 = srcv[b, pl.ds(g * 16, 16)]
                    d16 = dstv[b, pl.ds(g * 16, 16)]
                    hs16 = plsc.load_gather(hsv, [s16])
                    hd16 = plsc.load_gather(hdv, [d16])
                    e = hs16 + hd16
                    e = jnp.where(e >= 0.0, e, 0.2 * e)
                    eid = b * B + g * 16 + lane
                    ex = jnp.where(eid < EPS, jnp.exp(e), 0.0)
                    exv[pl.ds(g * 16, 16)] = ex
                    for u in range(16):
                        k = g * 16 + u
                        exk = plsc.load_gather(
                            exv, [jnp.full((16,), k, jnp.int32)])
                        for cc in range(DC // 16):
                            rbuf[k, pl.ds(cc * 16, 16)] = (
                                gbuf[k, pl.ds(cc * 16, 16)] * exk)
                        rbuf[k, pl.ds(DC, 16)] = jnp.where(
                            lane == 0, exk, 0.0)
                    return cy
                lax.fori_loop(0, B // 16, grpf, 0)

            issue_g(0, grows0, semg0)

            def pairf(p, carry):
                b0 = 2 * p
                b1 = b0 + 1
                wait_g(b0, grows0, semg0)
                issue_g(b1, grows1, semg1)

                @pl.when(p > 0)
                def _():
                    wait_s(b0 - 2, rows0, sems0)
                compute(b0, grows0, rows0)
                issue_s(b0, rows0, sems0)

                wait_g(b1, grows1, semg1)
                issue_g(b0 + 2, grows0, semg0)

                @pl.when(p > 0)
                def _():
                    wait_s(b1 - 2, rows1, sems1)
                compute(b1, grows1, rows1)
                issue_s(b1, rows1, sems1)
                return carry
            lax.fori_loop(0, (NBLK - 1) // 2, pairf, 0)

            bt = NBLK - 1
            wait_g(bt, grows0, semg0)
            wait_s(bt - 2, rows0, sems0)
            compute(bt, grows0, rows0)
            issue_s(bt, rows0, sems0)
            wait_s(bt, rows0, sems0)
            wait_s(bt - 1, rows1, sems1)

        plsc.subcore_barrier()

        def wb(j, carry):
            r0 = s * RPT + j * B
            pltpu.sync_copy(acc.at[pl.ds(r0, B)],
                            out_hbm.at[c, pl.ds(r0, B)])
            return carry
        lax.fori_loop(0, RPT // B, wb, 0)

    return sc_edge


_sc_edge_csplit = _make_sc_edge(True)
_sc_edge_esplit = _make_sc_edge(False)


def _tc_first(x, W, a_s, a_d):
    n, _ = x.shape
    D = W.shape[1]

    def body(x_ref, w_ref, as_ref, ad_ref,
             h_ref, hs_ref, hd_ref, ns_ref, ds_ref):
        h = jnp.dot(x_ref[...], w_ref[...], preferred_element_type=jnp.float32)
        hs = jnp.dot(h, as_ref[...], preferred_element_type=jnp.float32)
        hd = jnp.dot(h, ad_ref[...], preferred_element_type=jnp.float32)
        e = hs + hd
        e = jnp.where(e >= 0.0, e, 0.2 * e)
        exs = jnp.exp(e)
        h_ref[0] = h[:, :DC]
        h_ref[1] = h[:, DC:]
        hs_ref[...] = hs
        hd_ref[...] = hd
        ns_ref[...] = exs * h
        ds_ref[...] = exs

    return pl.pallas_call(
        body,
        out_shape=(
            jax.ShapeDtypeStruct((2, n, DC), jnp.float32),
            jax.ShapeDtypeStruct((n, 1), jnp.float32),
            jax.ShapeDtypeStruct((n, 1), jnp.float32),
            jax.ShapeDtypeStruct((n, D), jnp.float32),
            jax.ShapeDtypeStruct((n, 1), jnp.float32),
        ),
    )(x, W, a_s, a_d)


def _combine(part_ref, nums_ref, dens_ref, n, Dp, col_split):
    if col_split:
        num = (jnp.concatenate(
            [part_ref[0, :n, :DC], part_ref[1, :n, :DC]], axis=1)
            + nums_ref[...])
        den = part_ref[0, :n, DC:DC + 1] + dens_ref[...]
    else:
        num = part_ref[0, :n, :Dp] + part_ref[1, :n, :Dp] + nums_ref[...]
        den = (part_ref[0, :n, Dp:Dp + 1] + part_ref[1, :n, Dp:Dp + 1]
               + dens_ref[...])
    return num, den


def _tc_mid(part, nums, dens, b_prev, W, a_s, a_d, col_split):
    n, Dp = nums.shape
    D = W.shape[1]

    def body(part_ref, nums_ref, dens_ref, b_ref, w_ref, as_ref, ad_ref,
             h_ref, hs_ref, hd_ref, ns_ref, ds_ref):
        num, den = _combine(part_ref, nums_ref, dens_ref, n, Dp, col_split)
        prev = num / (den + 1e-16) + b_ref[...]
        prev = jnp.maximum(prev, 0.0)
        h = jnp.dot(prev, w_ref[...], preferred_element_type=jnp.float32)
        hs = jnp.dot(h, as_ref[...], preferred_element_type=jnp.float32)
        hd = jnp.dot(h, ad_ref[...], preferred_element_type=jnp.float32)
        e = hs + hd
        e = jnp.where(e >= 0.0, e, 0.2 * e)
        exs = jnp.exp(e)
        h_ref[...] = h
        hs_ref[...] = hs
        hd_ref[...] = hd
        ns_ref[...] = exs * h
        ds_ref[...] = exs

    return pl.pallas_call(
        body,
        out_shape=(
            jax.ShapeDtypeStruct((n, D), jnp.float32),
            jax.ShapeDtypeStruct((n, 1), jnp.float32),
            jax.ShapeDtypeStruct((n, 1), jnp.float32),
            jax.ShapeDtypeStruct((n, D), jnp.float32),
            jax.ShapeDtypeStruct((n, 1), jnp.float32),
        ),
    )(part, nums, dens, b_prev, W, a_s, a_d)


def _tc_final(part, nums, dens, b, td, cs):
    n, Dp = nums.shape

    def body(part_ref, nums_ref, dens_ref, b_ref, td_ref, cs_ref, out_ref):
        num, den = _combine(part_ref, nums_ref, dens_ref, n, Dp, False)
        o = num / (den + 1e-16) + b_ref[...]
        o = o * (td_ref[0, 0] * cs_ref[0, 0])
        out_ref[...] = jnp.mean(o, axis=0, keepdims=True)

    return pl.pallas_call(
        body,
        out_shape=jax.ShapeDtypeStruct((1, Dp), jnp.float32),
    )(part, nums, dens, b, td, cs)


def kernel(x, edge_index, W1, a1_src, a1_dst, b1, W2, a2_src, a2_dst, b2,
           W3, a3_src, a3_dst, b3, threshold_distance, cansu):
    pad = NBLK * B - EPS
    src = jnp.pad(edge_index[0].reshape(NSLAB, EPS),
                  ((0, 0), (0, pad))).reshape(NSLAB, NBLK, B)
    dst = jnp.pad(edge_index[1].reshape(NSLAB, EPS),
                  ((0, 0), (0, pad))).reshape(NSLAB, NBLK, B)
    td = threshold_distance.reshape(1, 1)
    cs = cansu.reshape(1, 1)

    h1, hs1, hd1, ns1, ds1 = _tc_first(
        x, W1, a1_src.reshape(-1, 1), a1_dst.reshape(-1, 1))
    part1 = _sc_edge_csplit(h1, hs1.reshape(-1), hd1.reshape(-1), src, dst)

    h2, hs2, hd2, ns2, ds2 = _tc_mid(
        part1, ns1, ds1, b1.reshape(1, -1),
        W2, a2_src.reshape(-1, 1), a2_dst.reshape(-1, 1), col_split=True)
    part2 = _sc_edge_esplit(h2, hs2.reshape(-1), hd2.reshape(-1), src, dst)

    h3, hs3, hd3, ns3, ds3 = _tc_mid(
        part2, ns2, ds2, b2.reshape(1, -1),
        W3, a3_src.reshape(-1, 1), a3_dst.reshape(-1, 1), col_split=False)
    part3 = _sc_edge_esplit(h3, hs3.reshape(-1), hd3.reshape(-1), src, dst)

    return _tc_final(part3, ns3, ds3, b3.reshape(1, -1), td, cs)


# speedup vs baseline: 1.9585x; 1.7910x over previous
import functools

import jax
import jax.numpy as jnp
from jax import lax
from jax.experimental import pallas as pl
from jax.experimental.pallas import tpu as pltpu
from jax.experimental.pallas import tpu_sc as plsc

N = 10000
E = 320000
NSLAB = 32
B = 80
NBLK = 125
EPS = 10000
ACC_N = 10240
RPT = ACC_N // 16
DC = 64
WA = DC + 16

_mesh = plsc.VectorSubcoreMesh(core_axis_name="c", subcore_axis_name="s")
_params = pltpu.CompilerParams(
    needs_layout_passes=False, use_tc_tiling_on_sc=False)


def _make_sc_edge(col_split):
    @functools.partial(
        pl.kernel,
        out_type=jax.ShapeDtypeStruct((2, ACC_N, WA), jnp.float32),
        mesh=_mesh,
        compiler_params=_params,
        scratch_types=[
            pltpu.VMEM((NBLK, B), jnp.int32),
            pltpu.VMEM((NBLK, B), jnp.int32),
            pltpu.VMEM((N,), jnp.float32),
            pltpu.VMEM((N,), jnp.float32),
            pltpu.VMEM((B, DC), jnp.float32),
            pltpu.VMEM((B, DC), jnp.float32),
            pltpu.VMEM((B, WA), jnp.float32),
            pltpu.VMEM((B, WA), jnp.float32),
            pltpu.VMEM((B,), jnp.float32),
            pltpu.VMEM_SHARED((ACC_N, WA), jnp.float32),
            pltpu.SemaphoreType.DMA,
            pltpu.SemaphoreType.DMA,
            pltpu.SemaphoreType.DMA,
            pltpu.SemaphoreType.DMA,
        ],
    )
    def sc_edge(ht_hbm, hs_hbm, hd_hbm, src_hbm, dst_hbm, out_hbm,
                srcv, dstv, hsv, hdv, grows0, grows1, rows0, rows1, exv,
                acc, semg0, semg1, sems0, sems1):
        c = lax.axis_index("c")
        s = lax.axis_index("s")

        pltpu.sync_copy(hs_hbm, hsv)
        pltpu.sync_copy(hd_hbm, hdv)

        z16 = jnp.zeros((16,), jnp.float32)
        cpr = WA // 16

        def zb(i, carry):
            r = i // cpr
            cc = i % cpr
            rows0[r, pl.ds(cc * 16, 16)] = z16
            return carry
        lax.fori_loop(0, B * cpr, zb, 0)

        def zcp(j, carry):
            pltpu.sync_copy(rows0, acc.at[pl.ds(s * RPT + j * B, B)])
            return carry
        lax.fori_loop(0, RPT // B, zcp, 0)

        plsc.subcore_barrier()

        lane = lax.iota(jnp.int32, 16)
        nslab_per_tile = 2 if col_split else 1

        for j in range(nslab_per_tile):
            if col_split:
                slab = s * 2 + j
                hsrc = ht_hbm.at[c]
            else:
                slab = c * 16 + s
                hsrc = ht_hbm
            pltpu.sync_copy(src_hbm.at[slab], srcv)
            pltpu.sync_copy(dst_hbm.at[slab], dstv)

            def issue_g(b, gbuf, sem):
                pltpu.async_copy(hsrc.at[srcv.at[b]], gbuf, sem)

            def wait_g(b, gbuf, sem):
                pltpu.make_async_copy(hsrc.at[srcv.at[b]], gbuf, sem).wait()

            def issue_s(b, rbuf, sem):
                pltpu.async_copy(rbuf, acc.at[dstv.at[b]], sem, add=True)

            def wait_s(b, rbuf, sem):
                pltpu.make_async_copy(rbuf, acc.at[dstv.at[b]], sem).wait()

            def compute(b, gbuf, rbuf):
                @functools.partial(plsc.parallel_loop, 0, B // 16)
                def _(g):
                    s16 = srcv[b, pl.ds(g * 16, 16)]
                    d16 = dstv[b, pl.ds(g * 16, 16)]
                    hs16 = plsc.load_gather(hsv, [s16])
                    hd16 = plsc.load_gather(hdv, [d16])
                    e = hs16 + hd16
                    e = jnp.where(e >= 0.0, e, 0.2 * e)
                    eid = b * B + g * 16 + lane
                    ex = jnp.where(eid < EPS, jnp.exp(e), 0.0)
                    for u in range(16):
                        k = g * 16 + u
                        exk = ex.at[jnp.full((16,), u, jnp.int32)].get(
                            mode="promise_in_bounds")
                        for cc in range(DC // 16):
                            rbuf[k, pl.ds(cc * 16, 16)] = (
                                gbuf[k, pl.ds(cc * 16, 16)] * exk)
                        rbuf[k, pl.ds(DC, 16)] = jnp.where(
                            lane == 0, exk, 0.0)

            issue_g(0, grows0, semg0)

            def pairf(p, carry):
                b0 = 2 * p
                b1 = b0 + 1
                wait_g(b0, grows0, semg0)
                issue_g(b1, grows1, semg1)

                @pl.when(p > 0)
                def _():
                    wait_s(b0 - 2, rows0, sems0)
                compute(b0, grows0, rows0)
                issue_s(b0, rows0, sems0)

                wait_g(b1, grows1, semg1)
                issue_g(b0 + 2, grows0, semg0)

                @pl.when(p > 0)
                def _():
                    wait_s(b1 - 2, rows1, sems1)
                compute(b1, grows1, rows1)
                issue_s(b1, rows1, sems1)
                return carry
            lax.fori_loop(0, (NBLK - 1) // 2, pairf, 0)

            bt = NBLK - 1
            wait_g(bt, grows0, semg0)
            wait_s(bt - 2, rows0, sems0)
            compute(bt, grows0, rows0)
            issue_s(bt, rows0, sems0)
            wait_s(bt, rows0, sems0)
            wait_s(bt - 1, rows1, sems1)

        plsc.subcore_barrier()

        def wb(j, carry):
            r0 = s * RPT + j * B
            pltpu.sync_copy(acc.at[pl.ds(r0, B)],
                            out_hbm.at[c, pl.ds(r0, B)])
            return carry
        lax.fori_loop(0, RPT // B, wb, 0)

    return sc_edge


_sc_edge_csplit = _make_sc_edge(True)
_sc_edge_esplit = _make_sc_edge(False)


def _tc_first(x, W, a_s, a_d):
    n, _ = x.shape
    D = W.shape[1]

    def body(x_ref, w_ref, as_ref, ad_ref,
             h_ref, hs_ref, hd_ref, ns_ref, ds_ref):
        h = jnp.dot(x_ref[...], w_ref[...], preferred_element_type=jnp.float32)
        hs = jnp.dot(h, as_ref[...], preferred_element_type=jnp.float32)
        hd = jnp.dot(h, ad_ref[...], preferred_element_type=jnp.float32)
        e = hs + hd
        e = jnp.where(e >= 0.0, e, 0.2 * e)
        exs = jnp.exp(e)
        h_ref[0] = h[:, :DC]
        h_ref[1] = h[:, DC:]
        hs_ref[...] = hs
        hd_ref[...] = hd
        ns_ref[...] = exs * h
        ds_ref[...] = exs

    return pl.pallas_call(
        body,
        out_shape=(
            jax.ShapeDtypeStruct((2, n, DC), jnp.float32),
            jax.ShapeDtypeStruct((n, 1), jnp.float32),
            jax.ShapeDtypeStruct((n, 1), jnp.float32),
            jax.ShapeDtypeStruct((n, D), jnp.float32),
            jax.ShapeDtypeStruct((n, 1), jnp.float32),
        ),
    )(x, W, a_s, a_d)


def _combine(part_ref, nums_ref, dens_ref, n, Dp, col_split):
    if col_split:
        num = (jnp.concatenate(
            [part_ref[0, :n, :DC], part_ref[1, :n, :DC]], axis=1)
            + nums_ref[...])
        den = part_ref[0, :n, DC:DC + 1] + dens_ref[...]
    else:
        num = part_ref[0, :n, :Dp] + part_ref[1, :n, :Dp] + nums_ref[...]
        den = (part_ref[0, :n, Dp:Dp + 1] + part_ref[1, :n, Dp:Dp + 1]
               + dens_ref[...])
    return num, den


def _tc_mid(part, nums, dens, b_prev, W, a_s, a_d, col_split):
    n, Dp = nums.shape
    D = W.shape[1]

    def body(part_ref, nums_ref, dens_ref, b_ref, w_ref, as_ref, ad_ref,
             h_ref, hs_ref, hd_ref, ns_ref, ds_ref):
        num, den = _combine(part_ref, nums_ref, dens_ref, n, Dp, col_split)
        prev = num / (den + 1e-16) + b_ref[...]
        prev = jnp.maximum(prev, 0.0)
        h = jnp.dot(prev, w_ref[...], preferred_element_type=jnp.float32)
        hs = jnp.dot(h, as_ref[...], preferred_element_type=jnp.float32)
        hd = jnp.dot(h, ad_ref[...], preferred_element_type=jnp.float32)
        e = hs + hd
        e = jnp.where(e >= 0.0, e, 0.2 * e)
        exs = jnp.exp(e)
        h_ref[...] = h
        hs_ref[...] = hs
        hd_ref[...] = hd
        ns_ref[...] = exs * h
        ds_ref[...] = exs

    return pl.pallas_call(
        body,
        out_shape=(
            jax.ShapeDtypeStruct((n, D), jnp.float32),
            jax.ShapeDtypeStruct((n, 1), jnp.float32),
            jax.ShapeDtypeStruct((n, 1), jnp.float32),
            jax.ShapeDtypeStruct((n, D), jnp.float32),
            jax.ShapeDtypeStruct((n, 1), jnp.float32),
        ),
    )(part, nums, dens, b_prev, W, a_s, a_d)


def _tc_final(part, nums, dens, b, td, cs):
    n, Dp = nums.shape

    def body(part_ref, nums_ref, dens_ref, b_ref, td_ref, cs_ref, out_ref):
        num, den = _combine(part_ref, nums_ref, dens_ref, n, Dp, False)
        o = num / (den + 1e-16) + b_ref[...]
        o = o * (td_ref[0, 0] * cs_ref[0, 0])
        out_ref[...] = jnp.mean(o, axis=0, keepdims=True)

    return pl.pallas_call(
        body,
        out_shape=jax.ShapeDtypeStruct((1, Dp), jnp.float32),
    )(part, nums, dens, b, td, cs)


def kernel(x, edge_index, W1, a1_src, a1_dst, b1, W2, a2_src, a2_dst, b2,
           W3, a3_src, a3_dst, b3, threshold_distance, cansu):
    pad = NBLK * B - EPS
    src = jnp.pad(edge_index[0].reshape(NSLAB, EPS),
                  ((0, 0), (0, pad))).reshape(NSLAB, NBLK, B)
    dst = jnp.pad(edge_index[1].reshape(NSLAB, EPS),
                  ((0, 0), (0, pad))).reshape(NSLAB, NBLK, B)
    td = threshold_distance.reshape(1, 1)
    cs = cansu.reshape(1, 1)

    h1, hs1, hd1, ns1, ds1 = _tc_first(
        x, W1, a1_src.reshape(-1, 1), a1_dst.reshape(-1, 1))
    part1 = _sc_edge_csplit(h1, hs1.reshape(-1), hd1.reshape(-1), src, dst)

    h2, hs2, hd2, ns2, ds2 = _tc_mid(
        part1, ns1, ds1, b1.reshape(1, -1),
        W2, a2_src.reshape(-1, 1), a2_dst.reshape(-1, 1), col_split=True)
    part2 = _sc_edge_esplit(h2, hs2.reshape(-1), hd2.reshape(-1), src, dst)

    h3, hs3, hd3, ns3, ds3 = _tc_mid(
        part2, ns2, ds2, b2.reshape(1, -1),
        W3, a3_src.reshape(-1, 1), a3_dst.reshape(-1, 1), col_split=False)
    part3 = _sc_edge_esplit(h3, hs3.reshape(-1), hd3.reshape(-1), src, dst)

    return _tc_final(part3, ns3, ds3, b3.reshape(1, -1), td, cs)
